# baseline (speedup 1.0000x reference)
import functools

import jax
import jax.numpy as jnp
from jax import lax
from jax.experimental import pallas as pl
from jax.experimental.pallas import tpu as pltpu
from jax.experimental.pallas import tpu_sc as plsc

NC = 2
NS = 16
NW = NC * NS
LANES = 16
K = 40
SB = 50
RD = 5


def _sc_segment_sum(T, N, D, E, C):
    DR_TILES = 10
    RT = N // DR_TILES
    CNT_R = N // DR_TILES
    NB = C // SB
    assert C % SB == 0 and SB % RD == 0
    KP = -(-K // LANES) * LANES

    mesh = plsc.VectorSubcoreMesh(
        core_axis_name="c", subcore_axis_name="s",
        num_cores=NC, num_subcores=NS)

    @functools.partial(
        pl.kernel,
        out_type=[
            jax.ShapeDtypeStruct((NC, T, N, D), jnp.float32),
            jax.ShapeDtypeStruct((NC, DR_TILES, 1, CNT_R), jnp.float32),
        ],
        mesh=mesh,
        scratch_types=[
            pltpu.VMEM((SB, K), jnp.int32),
            pltpu.VMEM((SB, K), jnp.int32),
            pltpu.VMEM((K, D), jnp.float32),
            pltpu.VMEM((K, D), jnp.float32),
            pltpu.VMEM((K, D), jnp.float32),
            pltpu.VMEM((K, D), jnp.float32),
            pltpu.VMEM((K, D), jnp.float32),
            pltpu.VMEM((KP,), jnp.float32),
            pltpu.VMEM((1, N // 10), jnp.float32),
            pltpu.VMEM_SHARED((N, D), jnp.float32),
            pltpu.VMEM_SHARED((N,), jnp.float32),
            pltpu.SemaphoreType.DMA,
            pltpu.SemaphoreType.DMA,
        ],
    )
    def sc_kernel(srcx_hbm, dst_hbm, x_hbm, z2d_hbm,
                  osum_hbm, ocnt_hbm,
                  src_v, dst_v, rb0, rb1, rb2, rb3, rb4, ones_v, cbuf_v,
                  acc_s, cnt_s, sem0, sem1):
        c = lax.axis_index("c")
        s = lax.axis_index("s")
        wid = s * NC + c

        ones16 = jnp.ones((LANES,), jnp.float32)
        zero16 = jnp.zeros((LANES,), jnp.float32)

        def init_ones(i, carry):
            ones_v[pl.ds(i * LANES, LANES)] = ones16
            return carry
        lax.fori_loop(0, KP // LANES, init_ones, 0)

        def init_cbuf(i, carry):
            cbuf_v[0, pl.ds(i * LANES, LANES)] = zero16
            return carry
        lax.fori_loop(0, CNT_R // LANES, init_cbuf, 0)

        @pl.when(s < DR_TILES)
        def _zero_cnt():
            pltpu.sync_copy(cbuf_v.at[0], cnt_s.at[pl.ds(s * CNT_R, CNT_R)])

        plsc.subcore_barrier()

        def cnt_block(b, carry):
            pltpu.sync_copy(dst_hbm.at[wid * NB + b], dst_v)

            def cnt_chunk(j, carry2):
                pltpu.sync_copy(ones_v.at[pl.ds(0, K)],
                                cnt_s.at[dst_v.at[j]], add=True)
                return carry2
            lax.fori_loop(0, SB, cnt_chunk, 0)
            return carry
        lax.fori_loop(0, NB, cnt_block, 0)

        plsc.subcore_barrier()

        @pl.when(s < DR_TILES)
        def _drain_cnt():
            pltpu.sync_copy(cnt_s.at[pl.ds(s * CNT_R, CNT_R)], cbuf_v.at[0])
            pltpu.sync_copy(cbuf_v, ocnt_hbm.at[c, s])

        def t_body(t, carry):
            @pl.when(s < DR_TILES)
            def _zero_acc():
                pltpu.sync_copy(z2d_hbm, acc_s.at[pl.ds(s * RT, RT)])
            plsc.subcore_barrier()

            def block(b, carry2):
                g = (t * NW + wid) * NB + b
                pltpu.sync_copy(srcx_hbm.at[g], src_v)
                pltpu.sync_copy(dst_hbm.at[wid * NB + b], dst_v)

                rbs = (rb0, rb1, rb2, rb3, rb4)

                def group(gi, carry3):
                    j0 = gi * RD
                    for r in range(RD):
                        pltpu.async_copy(
                            x_hbm.at[src_v.at[j0 + r]], rbs[r], sem0)
                    for r in range(RD):
                        pltpu.make_async_copy(
                            x_hbm.at[pl.ds(0, K)], rbs[r], sem0).wait()
                    for r in range(RD):
                        pltpu.sync_copy(rbs[r], acc_s.at[dst_v.at[j0 + r]],
                                        add=True)
                    return carry3
                lax.fori_loop(0, SB // RD, group, 0)
                return carry2
            lax.fori_loop(0, NB, block, 0)

            plsc.subcore_barrier()

            @pl.when(s < DR_TILES)
            def _drain_acc():
                r0 = s * RT
                pltpu.sync_copy(acc_s.at[pl.ds(r0, RT)],
                                osum_hbm.at[c, t, pl.ds(r0, RT)])
            return carry

        lax.fori_loop(0, T, t_body, 0)

    return sc_kernel


def _tc_finish(T, N, D, BN):
    grid = (T, N // BN)

    def body(cnt_ref, p_ref, x_ref, wl_ref, bl_ref, wr_ref, o_ref):
        cnt = cnt_ref[:, 0] + cnt_ref[:, 1]
        ssum = p_ref[0, 0] + p_ref[1, 0]
        mean = ssum / jnp.clip(cnt, 1.0, None)[:, None]
        h = (jnp.dot(mean, wl_ref[...], preferred_element_type=jnp.float32)
             + jnp.dot(x_ref[0], wr_ref[...],
                       preferred_element_type=jnp.float32)
             + bl_ref[0][None, :])
        o_ref[0] = h * 0.5 * (1.0 + lax.erf(h * 0.7071067811865476))

    return pl.pallas_call(
        body,
        grid=grid,
        in_specs=[
            pl.BlockSpec((BN, NC), lambda t, n: (n, 0)),
            pl.BlockSpec((NC, 1, BN, D), lambda t, n: (0, t, n, 0)),
            pl.BlockSpec((1, BN, D), lambda t, n: (t, n, 0)),
            pl.BlockSpec((D, D), lambda t, n: (0, 0)),
            pl.BlockSpec((1, D), lambda t, n: (0, 0)),
            pl.BlockSpec((D, D), lambda t, n: (0, 0)),
        ],
        out_specs=pl.BlockSpec((1, BN, D), lambda t, n: (t, n, 0)),
        out_shape=jax.ShapeDtypeStruct((T, N, D), jnp.float32),
    )


@jax.jit
def kernel(x, edge_index, W_l, b_l, W_r):
    T, N, D = x.shape
    E = edge_index.shape[1]
    assert E % NW == 0 and (E // NW) % K == 0
    C = E // NW // K
    NB = C // SB

    src = edge_index[0].reshape(1, NW, NB, SB, K)
    offs = (jnp.arange(T, dtype=jnp.int32) * N).reshape(T, 1, 1, 1, 1)
    srcx = (src + offs).reshape(T * NW * NB, SB, K)
    dst = edge_index[1].reshape(NW * NB, SB, K)
    x_flat = x.reshape(T * N, D)
    RT = N // 10
    z2d = jnp.zeros((RT, D), jnp.float32)

    osum, ocnt = _sc_segment_sum(T, N, D, E, C)(srcx, dst, x_flat, z2d)
    out = _tc_finish(T, N, D, 2000)(ocnt.reshape(NC, N).T, osum, x, W_l,
                                    b_l.reshape(1, D), W_r)
    return out
